# baseline (speedup 1.0000x reference)
import math
import jax
import jax.numpy as jnp
from jax.experimental import pallas as pl
from jax.experimental.pallas import tpu as pltpu

_CAPACITY_FACTOR = 2.0
_MIN_CAPACITY = 4


def _capacity(s, e):
    c = math.floor(_CAPACITY_FACTOR * s / e)
    c += c % 2
    return max(c, _MIN_CAPACITY)


def _router_body(cap, blk, nsplit, x_ref, cw_hbm, tgt_ref,
                 carry_ref, cw_buf, cw_sems):
    i = pl.program_id(0)
    n = pl.num_programs(0)
    t, e = x_ref.shape
    slot = jax.lax.rem(i, 2)
    sub = blk // nsplit

    def cw_copy(step, k):
        sl = jax.lax.rem(step, 2)
        return pltpu.make_async_copy(
            cw_buf.at[sl, pl.ds(k * sub, sub)],
            cw_hbm.at[pl.ds(step * blk + k * sub, sub)],
            cw_sems.at[sl, k],
        )

    @pl.when(i >= 2)
    def _reclaim():
        for k in range(nsplit):
            cw_copy(i - 2, k).wait()

    x = x_ref[...]

    xmax = jnp.max(x, axis=1, keepdims=True)
    denom = jnp.sum(jnp.exp(x - xmax), axis=1, keepdims=True)
    weight = 1.0 / denom

    colid = jax.lax.broadcasted_iota(jnp.int32, (t, e), 1)
    first = jnp.min(jnp.where(x == xmax, colid, e), axis=1, keepdims=True)
    oh = (colid == first).astype(jnp.float32)

    ri = jax.lax.broadcasted_iota(jnp.int32, (t, t), 0)
    ci = jax.lax.broadcasted_iota(jnp.int32, (t, t), 1)
    tril = (ri >= ci).astype(jnp.float32)
    cums = jnp.dot(tril, oh, preferred_element_type=jnp.float32)

    @pl.when(i == 0)
    def _init():
        carry_ref[...] = jnp.zeros_like(carry_ref)

    carry = carry_ref[0:1, :]
    rank = jnp.sum(oh * (cums + carry), axis=1, keepdims=True) - 1.0
    carry_ref[0:1, :] = carry + cums[t - 1 : t, :]

    rankk = jnp.where(rank < cap, rank.astype(jnp.int32), -1)

    c_i = jax.lax.broadcasted_iota(jnp.int32, (t, cap), 1)
    rw = jnp.where(c_i == rankk, 1.0, 0.0)
    ew = oh * weight

    target = jnp.where(rankk >= 0, first * cap + rankk, -1)
    tgt_ref[...] = target.reshape(t)

    for k in range(nsplit):
        rows = pl.ds(k * sub, sub)
        cw_buf[slot, rows] = (
            ew[k * sub : (k + 1) * sub].reshape(sub, e, 1)
            * rw[k * sub : (k + 1) * sub].reshape(sub, 1, cap)
        )
        cw_copy(i, k).start()

    @pl.when(i == n - 1)
    def _drain():
        for k in range(nsplit):
            cw_copy(i - 1, k).wait()
            cw_copy(i, k).wait()


def kernel(inputs):
    s, e = inputs.shape
    cap = _capacity(s, e)
    blk = 256
    nsplit = 2
    grid = s // blk

    body = lambda *refs: _router_body(cap, blk, nsplit, *refs)
    cw, tgt = pl.pallas_call(
        body,
        grid=(grid,),
        in_specs=[pl.BlockSpec((blk, e), lambda i: (i, 0))],
        out_specs=[
            pl.BlockSpec(memory_space=pl.ANY),
            pl.BlockSpec((blk,), lambda i: (i,)),
        ],
        out_shape=[
            jax.ShapeDtypeStruct((s, e, cap), jnp.float32),
            jax.ShapeDtypeStruct((s,), jnp.int32),
        ],
        scratch_shapes=[
            pltpu.VMEM((8, e), jnp.float32),
            pltpu.VMEM((2, blk, e, cap), jnp.float32),
            pltpu.SemaphoreType.DMA((2, nsplit)),
        ],
    )(inputs.astype(jnp.float32))

    e_iota = jax.lax.broadcasted_iota(jnp.int32, (s, e, cap), 1)
    c_iota = jax.lax.broadcasted_iota(jnp.int32, (s, e, cap), 2)
    sm = (e_iota * cap + c_iota) == tgt[:, None, None]
    return cw, sm


# speedup vs baseline: 1.2122x; 1.2122x over previous
import math
import jax
import jax.numpy as jnp
from jax.experimental import pallas as pl
from jax.experimental.pallas import tpu as pltpu

_CAPACITY_FACTOR = 2.0
_MIN_CAPACITY = 4


def _capacity(s, e):
    c = math.floor(_CAPACITY_FACTOR * s / e)
    c += c % 2
    return max(c, _MIN_CAPACITY)


def _router_body(cap, blk, nsplit, x_ref, cw_hbm, tgt_ref,
                 carry_ref, cw_buf, cw_sems):
    i = pl.program_id(0)
    n = pl.num_programs(0)
    t, e = x_ref.shape
    slot = jax.lax.rem(i, 2)
    sub = blk // nsplit

    def cw_copy(step, k):
        sl = jax.lax.rem(step, 2)
        return pltpu.make_async_copy(
            cw_buf.at[sl, pl.ds(k * sub, sub)],
            cw_hbm.at[pl.ds(step * blk + k * sub, sub)],
            cw_sems.at[sl, k],
        )

    @pl.when(i >= 2)
    def _reclaim():
        for k in range(nsplit):
            cw_copy(i - 2, k).wait()

    x = x_ref[...]

    xmax = jnp.max(x, axis=1, keepdims=True)
    denom = jnp.sum(jnp.exp(x - xmax), axis=1, keepdims=True)
    weight = 1.0 / denom

    colid = jax.lax.broadcasted_iota(jnp.int32, (t, e), 1)
    first = jnp.min(jnp.where(x == xmax, colid, e), axis=1, keepdims=True)
    oh = (colid == first).astype(jnp.float32)

    ri = jax.lax.broadcasted_iota(jnp.int32, (t, t), 0)
    ci = jax.lax.broadcasted_iota(jnp.int32, (t, t), 1)
    tril = (ri >= ci).astype(jnp.float32)
    cums = jnp.dot(tril, oh, preferred_element_type=jnp.float32)

    @pl.when(i == 0)
    def _init():
        carry_ref[...] = jnp.zeros_like(carry_ref)

    carry = carry_ref[0:1, :]
    rank = jnp.sum(oh * (cums + carry), axis=1, keepdims=True) - 1.0
    carry_ref[0:1, :] = carry + cums[t - 1 : t, :]

    rankk = jnp.where(rank < cap, rank.astype(jnp.int32), -1)

    c_i = jax.lax.broadcasted_iota(jnp.int32, (t, cap), 1)
    rw = jnp.where(c_i == rankk, 1.0, 0.0)
    ew = oh * weight

    target = jnp.where(rankk >= 0, first * cap + rankk, -1)
    tgt_ref[...] = target.reshape(t)

    for k in range(nsplit):
        rows = pl.ds(k * sub, sub)
        cw_buf[slot, rows] = (
            ew[k * sub : (k + 1) * sub].reshape(sub, e, 1)
            * rw[k * sub : (k + 1) * sub].reshape(sub, 1, cap)
        )
        cw_copy(i, k).start()

    @pl.when(i == n - 1)
    def _drain():
        for k in range(nsplit):
            cw_copy(i - 1, k).wait()
            cw_copy(i, k).wait()


def kernel(inputs):
    s, e = inputs.shape
    cap = _capacity(s, e)
    blk = 256
    nsplit = 2
    grid = s // blk

    body = lambda *refs: _router_body(cap, blk, nsplit, *refs)
    cw, tgt = pl.pallas_call(
        body,
        grid=(grid,),
        in_specs=[pl.BlockSpec((blk, e), lambda i: (i, 0))],
        out_specs=[
            pl.BlockSpec(memory_space=pl.ANY),
            pl.BlockSpec((blk,), lambda i: (i,)),
        ],
        out_shape=[
            jax.ShapeDtypeStruct((s, e, cap), jnp.float32),
            jax.ShapeDtypeStruct((s,), jnp.int32),
        ],
        scratch_shapes=[
            pltpu.VMEM((8, e), jnp.float32),
            pltpu.VMEM((2, blk, e, cap), jnp.float32),
            pltpu.SemaphoreType.DMA((2, nsplit)),
        ],
    )(inputs.astype(jnp.float32))

    e_iota = jax.lax.broadcasted_iota(jnp.int32, (s, e, cap), 1)
    c_iota = jax.lax.broadcasted_iota(jnp.int32, (s, e, cap), 2)
    fi = jnp.where(tgt >= 0, tgt // cap, -1)
    rk = jnp.where(tgt >= 0, tgt % cap, -1)
    sm = (e_iota == fi[:, None, None]) & (c_iota == rk[:, None, None])
    return cw, sm


# speedup vs baseline: 1.2266x; 1.0118x over previous
import math
import jax
import jax.numpy as jnp
from jax.experimental import pallas as pl
from jax.experimental.pallas import tpu as pltpu

_CAPACITY_FACTOR = 2.0
_MIN_CAPACITY = 4


def _capacity(s, e):
    c = math.floor(_CAPACITY_FACTOR * s / e)
    c += c % 2
    return max(c, _MIN_CAPACITY)


def _router_body(cap, blk, nsplit, x_ref, cw_hbm, fi_ref, rk_ref,
                 carry_ref, cw_buf, cw_sems):
    i = pl.program_id(0)
    n = pl.num_programs(0)
    t, e = x_ref.shape
    slot = jax.lax.rem(i, 2)
    sub = blk // nsplit

    def cw_copy(step, k):
        sl = jax.lax.rem(step, 2)
        return pltpu.make_async_copy(
            cw_buf.at[sl, pl.ds(k * sub, sub)],
            cw_hbm.at[pl.ds(step * blk + k * sub, sub)],
            cw_sems.at[sl, k],
        )

    @pl.when(i >= 2)
    def _reclaim():
        for k in range(nsplit):
            cw_copy(i - 2, k).wait()

    x = x_ref[...]

    xmax = jnp.max(x, axis=1, keepdims=True)
    denom = jnp.sum(jnp.exp(x - xmax), axis=1, keepdims=True)
    weight = 1.0 / denom

    colid = jax.lax.broadcasted_iota(jnp.int32, (t, e), 1)
    first = jnp.min(jnp.where(x == xmax, colid, e), axis=1, keepdims=True)
    oh = (colid == first).astype(jnp.float32)

    ri = jax.lax.broadcasted_iota(jnp.int32, (t, t), 0)
    ci = jax.lax.broadcasted_iota(jnp.int32, (t, t), 1)
    tril = (ri >= ci).astype(jnp.float32)
    cums = jnp.dot(tril, oh, preferred_element_type=jnp.float32)

    @pl.when(i == 0)
    def _init():
        carry_ref[...] = jnp.zeros_like(carry_ref)

    carry = carry_ref[0:1, :]
    rank = jnp.sum(oh * (cums + carry), axis=1, keepdims=True) - 1.0
    carry_ref[0:1, :] = carry + cums[t - 1 : t, :]

    rankk = jnp.where(rank < cap, rank.astype(jnp.int32), -1)

    c_i = jax.lax.broadcasted_iota(jnp.int32, (t, cap), 1)
    rw = jnp.where(c_i == rankk, 1.0, 0.0)
    ew = oh * weight

    fi_ref[...] = first.reshape(t)
    rk_ref[...] = rankk.reshape(t)

    for k in range(nsplit):
        rows = pl.ds(k * sub, sub)
        cw_buf[slot, rows] = (
            ew[k * sub : (k + 1) * sub].reshape(sub, e, 1)
            * rw[k * sub : (k + 1) * sub].reshape(sub, 1, cap)
        )
        cw_copy(i, k).start()

    @pl.when(i == n - 1)
    def _drain():
        for k in range(nsplit):
            cw_copy(i - 1, k).wait()
            cw_copy(i, k).wait()


def kernel(inputs):
    s, e = inputs.shape
    cap = _capacity(s, e)
    blk = 256
    nsplit = 2
    grid = s // blk

    body = lambda *refs: _router_body(cap, blk, nsplit, *refs)
    cw, fi, rk = pl.pallas_call(
        body,
        grid=(grid,),
        in_specs=[pl.BlockSpec((blk, e), lambda i: (i, 0))],
        out_specs=[
            pl.BlockSpec(memory_space=pl.ANY),
            pl.BlockSpec((blk,), lambda i: (i,)),
            pl.BlockSpec((blk,), lambda i: (i,)),
        ],
        out_shape=[
            jax.ShapeDtypeStruct((s, e, cap), jnp.float32),
            jax.ShapeDtypeStruct((s,), jnp.int32),
            jax.ShapeDtypeStruct((s,), jnp.int32),
        ],
        scratch_shapes=[
            pltpu.VMEM((8, e), jnp.float32),
            pltpu.VMEM((2, blk, e, cap), jnp.float32),
            pltpu.SemaphoreType.DMA((2, nsplit)),
        ],
    )(inputs.astype(jnp.float32))

    e_iota = jax.lax.broadcasted_iota(jnp.int32, (s, e, cap), 1)
    c_iota = jax.lax.broadcasted_iota(jnp.int32, (s, e, cap), 2)
    sm = (e_iota == fi[:, None, None]) & (c_iota == rk[:, None, None])
    return cw, sm
